# baseline (speedup 1.0000x reference)
import jax
import jax.numpy as jnp
from jax.experimental import pallas as pl

_NC = 21
_ROWS_PER_TILE = 2000


def _decode_tile(x_ref, o_ref):
    x = x_ref[...]
    o_ref[:, 0:_NC] = x[:, 0:_NC]
    loc_xy = x[:, 21:23]
    loc_wh = x[:, 23:25]
    anc_xy = x[:, 25:27]
    anc_wh = x[:, 27:29]
    var_xy = x[:, 29:31]
    var_wh = x[:, 31:33]
    c = loc_xy * anc_wh * var_xy + anc_xy
    wh = jnp.exp(loc_wh * var_wh) * anc_wh
    cs = c * 512.0
    hs = wh * 256.0
    o_ref[:, 21:23] = cs - hs
    o_ref[:, 23:25] = cs + hs


def kernel(prediction):
    b, n, cin = prediction.shape
    rows = b * n
    flat = prediction.reshape(rows, cin)
    out = pl.pallas_call(
        _decode_tile,
        grid=(rows // _ROWS_PER_TILE,),
        in_specs=[pl.BlockSpec((_ROWS_PER_TILE, cin), lambda i: (i, 0))],
        out_specs=pl.BlockSpec((_ROWS_PER_TILE, _NC + 4), lambda i: (i, 0)),
        out_shape=jax.ShapeDtypeStruct((rows, _NC + 4), jnp.float32),
    )(flat)
    return out.reshape(b, n, _NC + 4)


# speedup vs baseline: 4.5192x; 4.5192x over previous
import jax
import jax.numpy as jnp
from jax.experimental import pallas as pl

_NC = 21


def _decode_tile(x_ref, o_ref):
    x = x_ref[0]
    o_ref[0, 0:_NC, :] = x[0:_NC, :]
    dxy = x[21:23, :]
    dwh = x[23:25, :]
    axy = x[25:27, :]
    awh = x[27:29, :]
    vxy = x[29:31, :]
    vwh = x[31:33, :]
    c = dxy * awh * vxy + axy
    wh = jnp.exp(dwh * vwh) * awh
    cs = c * 512.0
    hs = wh * 256.0
    o_ref[0, 21:23, :] = cs - hs
    o_ref[0, 23:25, :] = cs + hs


def kernel(prediction):
    b, n, cin = prediction.shape
    xt = prediction.transpose(0, 2, 1)
    outt = pl.pallas_call(
        _decode_tile,
        grid=(b,),
        in_specs=[pl.BlockSpec((1, cin, n), lambda i: (i, 0, 0))],
        out_specs=pl.BlockSpec((1, _NC + 4, n), lambda i: (i, 0, 0)),
        out_shape=jax.ShapeDtypeStruct((b, _NC + 4, n), jnp.float32),
    )(xt)
    return outt.transpose(0, 2, 1)


# speedup vs baseline: 21.1632x; 4.6829x over previous
import jax
import jax.numpy as jnp
from jax.experimental import pallas as pl

_NC = 21
_L = 2048


def _decode_tile(x_ref, o_ref):
    x = x_ref[...]
    o_ref[0:_NC] = x[0:_NC]
    dxy = x[21:23]
    dwh = x[23:25]
    axy = x[25:27]
    awh = x[27:29]
    vxy = x[29:31]
    vwh = x[31:33]
    c = dxy * awh * vxy + axy
    wh = jnp.exp(dwh * vwh) * awh
    cs = c * 512.0
    hs = wh * 256.0
    o_ref[21:23] = cs - hs
    o_ref[23:25] = cs + hs


def kernel(prediction):
    b, n, cin = prediction.shape
    xt = prediction.transpose(2, 0, 1)
    outt = pl.pallas_call(
        _decode_tile,
        grid=(pl.cdiv(n, _L),),
        in_specs=[pl.BlockSpec((cin, b, _L), lambda j: (0, 0, j))],
        out_specs=pl.BlockSpec((_NC + 4, b, _L), lambda j: (0, 0, j)),
        out_shape=jax.ShapeDtypeStruct((_NC + 4, b, n), jnp.float32),
    )(xt)
    return outt.transpose(1, 2, 0)


# speedup vs baseline: 22.1397x; 1.0461x over previous
import jax
import jax.numpy as jnp
from jax.experimental import pallas as pl

_NC = 21
_L = 4096


def _decode_tile(x_ref, o_ref):
    x = x_ref[...]
    o_ref[0:_NC] = x[0:_NC]
    dxy = x[21:23]
    dwh = x[23:25]
    axy = x[25:27]
    awh = x[27:29]
    vxy = x[29:31]
    vwh = x[31:33]
    c = dxy * awh * vxy + axy
    wh = jnp.exp(dwh * vwh) * awh
    cs = c * 512.0
    hs = wh * 256.0
    o_ref[21:23] = cs - hs
    o_ref[23:25] = cs + hs


def kernel(prediction):
    b, n, cin = prediction.shape
    xt = prediction.transpose(2, 0, 1)
    outt = pl.pallas_call(
        _decode_tile,
        grid=(pl.cdiv(n, _L),),
        in_specs=[pl.BlockSpec((cin, b, _L), lambda j: (0, 0, j))],
        out_specs=pl.BlockSpec((_NC + 4, b, _L), lambda j: (0, 0, j)),
        out_shape=jax.ShapeDtypeStruct((_NC + 4, b, n), jnp.float32),
    )(xt)
    return outt.transpose(1, 2, 0)
